# baseline (speedup 1.0000x reference)
import jax
import jax.numpy as jnp
from jax import lax
from jax.experimental import pallas as pl
from jax.experimental.pallas import tpu as pltpu
from jax.experimental.pallas import tpu_sc as plsc

HIDDEN = 128
NLAYERS = 2
BATCH = 16384
NTAB = 4
DIM = HIDDEN // NTAB
ROW = DIM * NLAYERS
V0 = 1000000

_info = plsc.get_sparse_core_info()
NC, NS = _info.num_cores, _info.num_subcores
NW = NC * NS
BPW = BATCH // NW
CHUNK = 64
NCHUNK = BPW // CHUNK

WAVE = 384
WMAX = V0 // WAVE - 1
VLIM = (WMAX + 1) * WAVE
NREM = V0 - VLIM
PIECE = 128


def _k1(sidx, pos, c0t, h0t, piece_out,
        sidx_v, pos_v, cwaves, hwaves, piece, sem, psem):
    wid = lax.axis_index("s") * NC + lax.axis_index("c")
    base = pl.multiple_of(wid * BPW, BPW)
    lanes = jax.lax.iota(jnp.int32, 16)
    pltpu.sync_copy(sidx.at[pl.ds(base, BPW)], sidx_v)
    for j in range(BPW // PIECE):
        pltpu.sync_copy(pos.at[pl.ds(base + j * PIECE, PIECE)], pos_v.at[j])

    def _drain_pref(pref):
        slot = lax.bitwise_and(pref, 1)
        pltpu.make_async_copy(c0t.at[:, pl.ds(0, WAVE)],
                              cwaves.at[slot], psem).wait()
        pltpu.make_async_copy(h0t.at[:, pl.ds(0, WAVE)],
                              hwaves.at[slot], psem).wait()

    def body(i, st):
        wv_loaded, pref = st
        lane = lax.bitwise_and(i, 15)
        v16 = sidx_v[pl.ds(lax.bitwise_and(i, -16), 16)]
        v = jnp.sum(jnp.where(lanes == lane, v16, 0))
        wv = jnp.minimum(lax.div(v, jnp.int32(WAVE)), WMAX)

        def on_change(_):
            @pl.when(pref >= 0)
            def _():
                _drain_pref(pref)

            @pl.when(wv != pref)
            def _():
                woff = pl.multiple_of(wv * WAVE, 128)
                slot = lax.bitwise_and(wv, 1)
                pltpu.sync_copy(c0t.at[:, pl.ds(woff, WAVE)], cwaves.at[slot])
                pltpu.sync_copy(h0t.at[:, pl.ds(woff, WAVE)], hwaves.at[slot])

            nxt = jnp.minimum(wv + 1, WMAX)

            @pl.when(nxt != wv)
            def _():
                noff = pl.multiple_of(nxt * WAVE, 128)
                nslot = lax.bitwise_and(nxt, 1)
                pltpu.async_copy(c0t.at[:, pl.ds(noff, WAVE)],
                                 cwaves.at[nslot], psem)
                pltpu.async_copy(h0t.at[:, pl.ds(noff, WAVE)],
                                 hwaves.at[nslot], psem)

            return wv, jnp.where(nxt != wv, nxt, jnp.int32(-1))

        def no_change(_):
            return wv_loaded, pref

        wv_loaded, pref = lax.cond(wv != wv_loaded, on_change, no_change, 0)

        ip = lax.rem(i, PIECE)
        cl = jnp.minimum(v - wv * WAVE, WAVE - 1)
        colv = lanes * 0 + cl
        slotv = lanes * 0 + lax.bitwise_and(wv, 1)
        for k in range(4):
            piece[ip, pl.ds(k * 16, 16)] = plsc.load_gather(
                cwaves, [slotv, lanes + k * 16, colv])
            piece[ip, pl.ds(64 + k * 16, 16)] = plsc.load_gather(
                hwaves, [slotv, lanes + k * 16, colv])

        @pl.when(ip == PIECE - 1)
        def _():
            j = lax.shift_right_logical(i, 7)
            pltpu.async_copy(piece, piece_out.at[pos_v.at[j]], sem).wait()

        return wv_loaded, pref

    _, pref = lax.fori_loop(0, BPW, body, (jnp.int32(-1), jnp.int32(-1)))

    @pl.when(pref >= 0)
    def _():
        _drain_pref(pref)


def _k2(idx0, idx1, idx2, idx3,
        c1, c2, c3, h1, h2, h3, c0r, h0r, piece_in,
        cs_out, hs_out,
        idx4v, rid3, cgbufs, hgbufs, cobuf, hobuf, crem, hrem, pbuf, sem):
    wid = lax.axis_index("s") * NC + lax.axis_index("c")
    base = pl.multiple_of(wid * BPW, BPW)
    idxs = (idx1, idx2, idx3)
    ctabs = (c1, c2, c3)
    htabs = (h1, h2, h3)
    lanes = jax.lax.iota(jnp.int32, 16)
    pltpu.sync_copy(c0r, crem)
    pltpu.sync_copy(h0r, hrem)

    def chunk_body(j, carry):
        cb = pl.multiple_of(base + j * CHUNK, CHUNK)
        pltpu.sync_copy(idx0.at[pl.ds(cb, CHUNK)], idx4v.at[0])
        handles = []
        for ti in range(3):
            pltpu.sync_copy(idxs[ti].at[pl.ds(cb, CHUNK)], idx4v.at[ti + 1])
            for g in range(CHUNK // 16):
                iv = idx4v[ti + 1, pl.ds(g * 16, 16)]
                rid3[ti, pl.ds(g * 16, 16)] = lax.shift_right_logical(iv, 1)
            handles.append(pltpu.async_copy(
                ctabs[ti].at[rid3.at[ti]], cgbufs.at[ti], sem))
            handles.append(pltpu.async_copy(
                htabs[ti].at[rid3.at[ti]], hgbufs.at[ti], sem))

        pltpu.sync_copy(piece_in.at[pl.ds(cb, CHUNK), :], pbuf)

        def merge(r, carry2):
            for si, so in ((0, cobuf), (1, hobuf)):
                for l in range(NLAYERS):
                    for q in range(2):
                        so[l * CHUNK + r, pl.ds(q * 16, 16)] = (
                            pbuf[r, pl.ds(si * 64 + l * DIM + q * 16, 16)])
            return carry2

        lax.fori_loop(0, CHUNK, merge, None)
        for h in handles:
            h.wait()

        def extract(g, carry2):
            v0_16 = idx4v[0, pl.ds(g * 16, 16)]
            off16 = []
            for ti in range(3):
                iv = idx4v[ti + 1, pl.ds(g * 16, 16)]
                off16.append(lax.shift_left(lax.bitwise_and(iv, 1), 6))
            for k in range(16):
                r = g * 16 + k
                msk = lanes == k
                for ti in range(3):
                    t = ti + 1
                    off = jnp.sum(jnp.where(msk, off16[ti], 0))
                    for l in range(NLAYERS):
                        src = off + l * DIM
                        for q in range(2):
                            cobuf[l * CHUNK + r, pl.ds(t * DIM + q * 16, 16)] = (
                                cgbufs[ti, r, pl.ds(src + q * 16, 16)])
                            hobuf[l * CHUNK + r, pl.ds(t * DIM + q * 16, 16)] = (
                                hgbufs[ti, r, pl.ds(src + q * 16, 16)])
                v0 = jnp.sum(jnp.where(msk, v0_16, 0))

                @pl.when(v0 >= VLIM)
                def _():
                    colv = lanes * 0 + (v0 - VLIM)
                    for l in range(NLAYERS):
                        for q in range(2):
                            feats = lanes + l * DIM + q * 16
                            cobuf[l * CHUNK + r, pl.ds(q * 16, 16)] = (
                                plsc.load_gather(crem, [feats, colv]))
                            hobuf[l * CHUNK + r, pl.ds(q * 16, 16)] = (
                                plsc.load_gather(hrem, [feats, colv]))

            return carry2

        lax.fori_loop(0, CHUNK // 16, extract, None)

        for l in range(NLAYERS):
            pltpu.sync_copy(cobuf.at[pl.ds(l * CHUNK, CHUNK)],
                            cs_out.at[l, pl.ds(cb, CHUNK), :])
            pltpu.sync_copy(hobuf.at[pl.ds(l * CHUNK, CHUNK)],
                            hs_out.at[l, pl.ds(cb, CHUNK), :])
        return carry

    lax.fori_loop(0, NCHUNK, chunk_body, None)


def kernel(idx0, idx1, idx2, idx3,
           c_emb0, c_emb1, c_emb2, c_emb3,
           h_emb0, h_emb1, h_emb2, h_emb3):
    mesh = plsc.VectorSubcoreMesh(core_axis_name="c", subcore_axis_name="s")
    params = pltpu.CompilerParams(needs_layout_passes=False)
    idx0 = idx0.astype(jnp.int32)
    idx1 = idx1.astype(jnp.int32)
    idx2 = idx2.astype(jnp.int32)
    idx3 = idx3.astype(jnp.int32)

    sidx, pos = lax.sort_key_val(idx0, jnp.arange(BATCH, dtype=jnp.int32))

    piece_t = jax.ShapeDtypeStruct((BATCH, HIDDEN), jnp.float32)
    run1 = pl.kernel(
        _k1, mesh=mesh, compiler_params=params,
        out_type=piece_t,
        scratch_types=[
            pltpu.VMEM((BPW,), jnp.int32),
            pltpu.VMEM((BPW // PIECE, PIECE), jnp.int32),
            pltpu.VMEM((2, ROW, WAVE), jnp.float32),
            pltpu.VMEM((2, ROW, WAVE), jnp.float32),
            pltpu.VMEM((PIECE, HIDDEN), jnp.float32),
            pltpu.SemaphoreType.DMA,
            pltpu.SemaphoreType.DMA,
        ],
    )
    piece = run1(sidx, pos, jnp.transpose(c_emb0), jnp.transpose(h_emb0))

    out_t = jax.ShapeDtypeStruct((NLAYERS, BATCH, HIDDEN), jnp.float32)
    run2 = pl.kernel(
        _k2, mesh=mesh, compiler_params=params,
        out_type=(out_t, out_t),
        scratch_types=[
            pltpu.VMEM((4, CHUNK), jnp.int32),
            pltpu.VMEM((3, CHUNK), jnp.int32),
            pltpu.VMEM((3, CHUNK, HIDDEN), jnp.float32),
            pltpu.VMEM((3, CHUNK, HIDDEN), jnp.float32),
            pltpu.VMEM((NLAYERS * CHUNK, HIDDEN), jnp.float32),
            pltpu.VMEM((NLAYERS * CHUNK, HIDDEN), jnp.float32),
            pltpu.VMEM((ROW, NREM), jnp.float32),
            pltpu.VMEM((ROW, NREM), jnp.float32),
            pltpu.VMEM((CHUNK, HIDDEN), jnp.float32),
            pltpu.SemaphoreType.DMA,
        ],
    )
    def _pairs(x):
        v, d = x.shape
        xt = jnp.transpose(x)
        return xt.reshape(d, v // 2, 2).transpose(1, 2, 0).reshape(v // 2,
                                                                   2 * d)
    def _resh(x):
        return x.reshape(x.shape[0] // 2, 2 * x.shape[1])
    mids = [_pairs(c_emb1), _resh(c_emb2), _resh(c_emb3),
            _pairs(h_emb1), _resh(h_emb2), _resh(h_emb3)]
    c0r = jnp.transpose(c_emb0[VLIM:])
    h0r = jnp.transpose(h_emb0[VLIM:])
    return run2(idx0, idx1, idx2, idx3, *mids, c0r, h0r, piece)


# speedup vs baseline: 1.0888x; 1.0888x over previous
import jax
import jax.numpy as jnp
from jax import lax
from jax.experimental import pallas as pl
from jax.experimental.pallas import tpu as pltpu
from jax.experimental.pallas import tpu_sc as plsc

HIDDEN = 128
NLAYERS = 2
BATCH = 16384
NTAB = 4
DIM = HIDDEN // NTAB
ROW = DIM * NLAYERS
V0 = 1000000

_info = plsc.get_sparse_core_info()
NC, NS = _info.num_cores, _info.num_subcores
NW = NC * NS
BPW = BATCH // NW
CHUNK = 64
NCHUNK = BPW // CHUNK

WAVE = 384
WMAX = V0 // WAVE - 1
VLIM = (WMAX + 1) * WAVE
NREM = V0 - VLIM
PIECE = 128


def _k1(sidx, pos, c0t, h0t, piece_out,
        sidx_v, pos_v, cwaves, hwaves, piece, sem, psem):
    wid = lax.axis_index("s") * NC + lax.axis_index("c")
    base = pl.multiple_of(wid * BPW, BPW)
    lanes = jax.lax.iota(jnp.int32, 16)
    pltpu.sync_copy(sidx.at[pl.ds(base, BPW)], sidx_v)
    for j in range(BPW // PIECE):
        pltpu.sync_copy(pos.at[pl.ds(base + j * PIECE, PIECE)], pos_v.at[j])

    def _drain_pref(pref):
        slot = lax.bitwise_and(pref, 1)
        pltpu.make_async_copy(c0t.at[:, pl.ds(0, WAVE)],
                              cwaves.at[slot], psem).wait()
        pltpu.make_async_copy(h0t.at[:, pl.ds(0, WAVE)],
                              hwaves.at[slot], psem).wait()

    def body(i, st):
        wv_loaded, pref = st
        lane = lax.bitwise_and(i, 15)
        v16 = sidx_v[pl.ds(lax.bitwise_and(i, -16), 16)]
        v = jnp.sum(jnp.where(lanes == lane, v16, 0))
        wv = jnp.minimum(lax.div(v, jnp.int32(WAVE)), WMAX)

        def on_change(_):
            @pl.when(pref >= 0)
            def _():
                _drain_pref(pref)

            @pl.when(wv != pref)
            def _():
                woff = pl.multiple_of(wv * WAVE, 128)
                slot = lax.bitwise_and(wv, 1)
                pltpu.sync_copy(c0t.at[:, pl.ds(woff, WAVE)], cwaves.at[slot])
                pltpu.sync_copy(h0t.at[:, pl.ds(woff, WAVE)], hwaves.at[slot])

            nxt = jnp.minimum(wv + 1, WMAX)

            @pl.when(nxt != wv)
            def _():
                noff = pl.multiple_of(nxt * WAVE, 128)
                nslot = lax.bitwise_and(nxt, 1)
                pltpu.async_copy(c0t.at[:, pl.ds(noff, WAVE)],
                                 cwaves.at[nslot], psem)
                pltpu.async_copy(h0t.at[:, pl.ds(noff, WAVE)],
                                 hwaves.at[nslot], psem)

            return wv, jnp.where(nxt != wv, nxt, jnp.int32(-1))

        def no_change(_):
            return wv_loaded, pref

        wv_loaded, pref = lax.cond(wv != wv_loaded, on_change, no_change, 0)

        ip = lax.rem(i, PIECE)
        cl = jnp.minimum(v - wv * WAVE, WAVE - 1)
        colv = lanes * 0 + cl
        slotv = lanes * 0 + lax.bitwise_and(wv, 1)
        for k in range(4):
            piece[ip, pl.ds(k * 16, 16)] = plsc.load_gather(
                cwaves, [slotv, lanes + k * 16, colv])
            piece[ip, pl.ds(64 + k * 16, 16)] = plsc.load_gather(
                hwaves, [slotv, lanes + k * 16, colv])

        @pl.when(ip == PIECE - 1)
        def _():
            j = lax.shift_right_logical(i, 7)
            pltpu.async_copy(piece, piece_out.at[pos_v.at[j]], sem).wait()

        return wv_loaded, pref

    _, pref = lax.fori_loop(0, BPW, body, (jnp.int32(-1), jnp.int32(-1)))

    @pl.when(pref >= 0)
    def _():
        _drain_pref(pref)


def _k2(idx0, idx1, idx2, idx3,
        c1, c2, c3, h1, h2, h3, c0r, h0r, piece_in,
        cs_out, hs_out,
        idx4v, rid3, cgbufs, hgbufs, cobuf, hobuf, crem, hrem, pbuf, sem):
    wid = lax.axis_index("s") * NC + lax.axis_index("c")
    base = pl.multiple_of(wid * BPW, BPW)
    idxs = (idx1, idx2, idx3)
    ctabs = (c1, c2, c3)
    htabs = (h1, h2, h3)
    lanes = jax.lax.iota(jnp.int32, 16)
    pltpu.sync_copy(c0r, crem)
    pltpu.sync_copy(h0r, hrem)

    def chunk_body(j, carry):
        cb = pl.multiple_of(base + j * CHUNK, CHUNK)
        pltpu.sync_copy(idx0.at[pl.ds(cb, CHUNK)], idx4v.at[0])
        handles = []
        for ti in range(3):
            pltpu.sync_copy(idxs[ti].at[pl.ds(cb, CHUNK)], idx4v.at[ti + 1])
            for g in range(CHUNK // 16):
                iv = idx4v[ti + 1, pl.ds(g * 16, 16)]
                rid3[ti, pl.ds(g * 16, 16)] = lax.shift_right_logical(iv, 1)
            handles.append(pltpu.async_copy(
                ctabs[ti].at[rid3.at[ti]], cgbufs.at[ti], sem))
            handles.append(pltpu.async_copy(
                htabs[ti].at[rid3.at[ti]], hgbufs.at[ti], sem))

        pltpu.sync_copy(piece_in.at[pl.ds(cb, CHUNK), :], pbuf)

        def merge(r, carry2):
            for si, so in ((0, cobuf), (1, hobuf)):
                for l in range(NLAYERS):
                    for q in range(2):
                        so[l * CHUNK + r, pl.ds(q * 16, 16)] = (
                            pbuf[r, pl.ds(si * 64 + l * DIM + q * 16, 16)])
            return carry2

        lax.fori_loop(0, CHUNK, merge, None)
        for h in handles:
            h.wait()

        def extract(g, carry2):
            v0_16 = idx4v[0, pl.ds(g * 16, 16)]
            off16 = []
            for ti in range(3):
                iv = idx4v[ti + 1, pl.ds(g * 16, 16)]
                off16.append(lax.shift_left(lax.bitwise_and(iv, 1), 6))
            for k in range(16):
                r = g * 16 + k
                msk = lanes == k
                for ti in range(3):
                    t = ti + 1
                    off = jnp.sum(jnp.where(msk, off16[ti], 0))
                    for l in range(NLAYERS):
                        src = off + l * DIM
                        for q in range(2):
                            cobuf[l * CHUNK + r, pl.ds(t * DIM + q * 16, 16)] = (
                                cgbufs[ti, r, pl.ds(src + q * 16, 16)])
                            hobuf[l * CHUNK + r, pl.ds(t * DIM + q * 16, 16)] = (
                                hgbufs[ti, r, pl.ds(src + q * 16, 16)])
                v0 = jnp.sum(jnp.where(msk, v0_16, 0))

                @pl.when(v0 >= VLIM)
                def _():
                    colv = lanes * 0 + (v0 - VLIM)
                    for l in range(NLAYERS):
                        for q in range(2):
                            feats = lanes + l * DIM + q * 16
                            cobuf[l * CHUNK + r, pl.ds(q * 16, 16)] = (
                                plsc.load_gather(crem, [feats, colv]))
                            hobuf[l * CHUNK + r, pl.ds(q * 16, 16)] = (
                                plsc.load_gather(hrem, [feats, colv]))

            return carry2

        lax.fori_loop(0, CHUNK // 16, extract, None)

        for l in range(NLAYERS):
            pltpu.sync_copy(cobuf.at[pl.ds(l * CHUNK, CHUNK)],
                            cs_out.at[l, pl.ds(cb, CHUNK), :])
            pltpu.sync_copy(hobuf.at[pl.ds(l * CHUNK, CHUNK)],
                            hs_out.at[l, pl.ds(cb, CHUNK), :])
        return carry

    lax.fori_loop(0, NCHUNK, chunk_body, None)


def kernel(idx0, idx1, idx2, idx3,
           c_emb0, c_emb1, c_emb2, c_emb3,
           h_emb0, h_emb1, h_emb2, h_emb3):
    mesh = plsc.VectorSubcoreMesh(core_axis_name="c", subcore_axis_name="s")
    params = pltpu.CompilerParams(needs_layout_passes=False)
    idx0 = idx0.astype(jnp.int32)
    idx1 = idx1.astype(jnp.int32)
    idx2 = idx2.astype(jnp.int32)
    idx3 = idx3.astype(jnp.int32)

    sidx, pos = lax.sort_key_val(idx0, jnp.arange(BATCH, dtype=jnp.int32))

    piece_t = jax.ShapeDtypeStruct((BATCH, HIDDEN), jnp.float32)
    run1 = pl.kernel(
        _k1, mesh=mesh, compiler_params=params,
        out_type=piece_t,
        scratch_types=[
            pltpu.VMEM((BPW,), jnp.int32),
            pltpu.VMEM((BPW // PIECE, PIECE), jnp.int32),
            pltpu.VMEM((2, ROW, WAVE), jnp.float32),
            pltpu.VMEM((2, ROW, WAVE), jnp.float32),
            pltpu.VMEM((PIECE, HIDDEN), jnp.float32),
            pltpu.SemaphoreType.DMA,
            pltpu.SemaphoreType.DMA,
        ],
    )
    piece = run1(sidx, pos, jnp.transpose(c_emb0), jnp.transpose(h_emb0))

    out_t = jax.ShapeDtypeStruct((NLAYERS, BATCH, HIDDEN), jnp.float32)
    run2 = pl.kernel(
        _k2, mesh=mesh, compiler_params=params,
        out_type=(out_t, out_t),
        scratch_types=[
            pltpu.VMEM((4, CHUNK), jnp.int32),
            pltpu.VMEM((3, CHUNK), jnp.int32),
            pltpu.VMEM((3, CHUNK, HIDDEN), jnp.float32),
            pltpu.VMEM((3, CHUNK, HIDDEN), jnp.float32),
            pltpu.VMEM((NLAYERS * CHUNK, HIDDEN), jnp.float32),
            pltpu.VMEM((NLAYERS * CHUNK, HIDDEN), jnp.float32),
            pltpu.VMEM((ROW, NREM), jnp.float32),
            pltpu.VMEM((ROW, NREM), jnp.float32),
            pltpu.VMEM((CHUNK, HIDDEN), jnp.float32),
            pltpu.SemaphoreType.DMA,
        ],
    )
    mids = [x.reshape(x.shape[0] // 2, 2 * x.shape[1])
            for x in (c_emb1, c_emb2, c_emb3, h_emb1, h_emb2, h_emb3)]
    c0r = jnp.transpose(c_emb0[VLIM:])
    h0r = jnp.transpose(h_emb0[VLIM:])
    return run2(idx0, idx1, idx2, idx3, *mids, c0r, h0r, piece)
